# Initial kernel scaffold; baseline (speedup 1.0000x reference)
import jax
import jax.numpy as jnp
from jax.experimental import pallas as pl


def kernel(x, edge_index, batch, W1, b1, W2, b2, W3, b3, wp, Wl1, bl1, Wl2, bl2, Wl3, bl3):
    raise NotImplementedError("write your pallas kernel here")


# speedup vs baseline: 18.0515x; 18.0515x over previous
import functools

import jax
import jax.numpy as jnp
from jax import lax
from jax.experimental import pallas as pl
from jax.experimental.pallas import tpu as pltpu
from jax.experimental.pallas import tpu_sc as plsc

_N = 10000
_E = 320000
_D = 128
_G = 100
_H = 32

_NC = 2
_NS = 16
_NW = _NC * _NS
_CH = 128
_NCHUNK = 79
_EPW = _NCHUNK * _CH
_EP = _NW * _EPW
_NP = 10240
_RPT = _NP // _NS
_BIG_NEG = -3.4e38

_HIGH = jax.lax.Precision.HIGHEST


def _mm(a, b):
    return lax.dot_general(a, b, (((1,), (0,)), ((), ())),
                           precision=_HIGH, preferred_element_type=jnp.float32)


def _mm_rt(a, b):
    return lax.dot_general(a, b, (((1,), (1,)), ((), ())),
                           precision=_HIGH, preferred_element_type=jnp.float32)


_MESH = plsc.VectorSubcoreMesh(core_axis_name="c", subcore_axis_name="s")


@functools.partial(
    pl.kernel,
    out_type=jax.ShapeDtypeStruct((_NC, _NP), jnp.float32),
    mesh=_MESH,
    compiler_params=pltpu.CompilerParams(use_tc_tiling_on_sc=False),
    scratch_types=[
        pltpu.VMEM((_NCHUNK, _CH), jnp.int32),
        pltpu.VMEM((_CH,), jnp.float32),
        pltpu.VMEM_SHARED((_NP,), jnp.float32),
    ],
)
def _deg_sc(dstw_hbm, ones_hbm, zeros_hbm, out_hbm, dst_v, ones_v, deg_sh):
    c = lax.axis_index("c")
    s = lax.axis_index("s")
    wid = s * _NC + c
    pltpu.sync_copy(ones_hbm, ones_v)
    pltpu.sync_copy(zeros_hbm.at[pl.ds(s * _RPT, _RPT)],
                    deg_sh.at[pl.ds(s * _RPT, _RPT)])
    pltpu.sync_copy(dstw_hbm.at[wid], dst_v)
    plsc.subcore_barrier()

    def chunk(j, carry):
        pltpu.sync_copy(ones_v, deg_sh.at[dst_v.at[j]], add=True)
        return carry

    lax.fori_loop(0, _NCHUNK, chunk, 0)
    plsc.subcore_barrier()
    pltpu.sync_copy(deg_sh.at[pl.ds(s * _RPT, _RPT)],
                    out_hbm.at[c, pl.ds(s * _RPT, _RPT)])


@functools.partial(
    pl.kernel,
    out_type=jax.ShapeDtypeStruct((_NC, _NP, _H), jnp.float32),
    mesh=_MESH,
    compiler_params=pltpu.CompilerParams(use_tc_tiling_on_sc=False),
    scratch_types=[
        pltpu.VMEM((_NCHUNK, _CH), jnp.int32),
        pltpu.VMEM((_NCHUNK, _CH), jnp.int32),
        pltpu.VMEM((_CH, _H), jnp.float32),
        pltpu.VMEM_SHARED((_NP, _H), jnp.float32),
        pltpu.SemaphoreType.DMA,
    ],
)
def _msg_sc(h_hbm, srcw_hbm, dstw_hbm, zeros2_hbm, out_hbm,
            src_v, dst_v, rows_v, acc_sh, sem):
    c = lax.axis_index("c")
    s = lax.axis_index("s")
    wid = s * _NC + c
    pltpu.sync_copy(zeros2_hbm.at[pl.ds(s * _RPT, _RPT)],
                    acc_sh.at[pl.ds(s * _RPT, _RPT)])
    pltpu.sync_copy(srcw_hbm.at[wid], src_v)
    pltpu.sync_copy(dstw_hbm.at[wid], dst_v)
    plsc.subcore_barrier()

    def chunk(j, carry):
        pltpu.async_copy(h_hbm.at[src_v.at[j]], rows_v, sem).wait()
        pltpu.sync_copy(rows_v, acc_sh.at[dst_v.at[j]], add=True)
        return carry

    lax.fori_loop(0, _NCHUNK, chunk, 0)
    plsc.subcore_barrier()
    pltpu.sync_copy(acc_sh.at[pl.ds(s * _RPT, _RPT)],
                    out_hbm.at[c, pl.ds(s * _RPT, _RPT)])


def _dinv_row(degp):
    return lax.rsqrt(1.0 + degp[0:1, :_N] + degp[1:2, :_N])


def _onehot_t(batr):
    gids = lax.broadcasted_iota(jnp.int32, (_G, _N), 0)
    oht = (batr == gids).astype(jnp.float32)
    ones = jnp.ones((1, _N), jnp.float32)
    cntr = _mm_rt(ones, oht)
    return oht, cntr


def _gmp_t(batr, vt, cntr):
    gcol = lax.broadcasted_iota(jnp.int32, (1, _G), 1)

    def body(g, carry):
        m = jnp.max(jnp.where(batr == g, vt, _BIG_NEG), axis=1, keepdims=True)
        return jnp.where(gcol == g, m, carry)

    mt = lax.fori_loop(0, _G, body,
                       jnp.full((_H, _G), _BIG_NEG, jnp.float32))
    return jnp.where(cntr > 0, mt, 0.0)


def _tc_pre1_body(xt_ref, batr_ref, degp_ref, wat_ref, wbt_ref, h1pt_ref):
    xt = xt_ref[...]
    batr = batr_ref[...]
    dinvr = _dinv_row(degp_ref[...])
    oht, cntr = _onehot_t(batr)
    s0t = _mm_rt(xt, oht)
    xgt = s0t / jnp.maximum(cntr, 1.0)
    hgt = _mm(wbt_ref[...], xgt)
    h1t = _mm(wat_ref[...], xt) + _mm(hgt, oht)
    h1pt_ref[...] = h1t * dinvr


def _tc_pre1(xt, batr, degp, wat, wbt):
    return pl.pallas_call(
        _tc_pre1_body,
        out_shape=jax.ShapeDtypeStruct((_H, _N), jnp.float32),
    )(xt, batr, degp, wat, wbt)


def _tc_mid_body(accpt_ref, hpt_ref, degp_ref, batr_ref, bc_ref,
                 wat_ref, wbt_ref, poolt_ref, hnextt_ref):
    batr = batr_ref[...]
    dinvr = _dinv_row(degp_ref[...])
    acc = accpt_ref[0, :, :_N] + accpt_ref[1, :, :_N] + hpt_ref[...]
    xot = jnp.maximum(dinvr * acc + bc_ref[...], 0.0)
    oht, cntr = _onehot_t(batr)
    gapt = _mm_rt(xot, oht) / jnp.maximum(cntr, 1.0)
    gmpt = _gmp_t(batr, xot, cntr)
    poolt_ref[...] = jnp.concatenate([gmpt, gapt], axis=0)
    hgt = _mm(wbt_ref[...], gapt)
    hnextt_ref[...] = (_mm(wat_ref[...], xot) + _mm(hgt, oht)) * dinvr


def _tc_mid(accpt, hpt, degp, batr, bc, wat, wbt):
    return pl.pallas_call(
        _tc_mid_body,
        out_shape=[
            jax.ShapeDtypeStruct((2 * _H, _G), jnp.float32),
            jax.ShapeDtypeStruct((_H, _N), jnp.float32),
        ],
    )(accpt, hpt, degp, batr, bc, wat, wbt)


def _tc_final_body(accpt_ref, hpt_ref, degp_ref, batr_ref, bc_ref, wpt_ref,
                   x1t_ref, x2t_ref, wl1t_ref, bl1c_ref, wl2t_ref, bl2c_ref,
                   wl3t_ref, bl3c_ref, out_ref):
    batr = batr_ref[...]
    dinvr = _dinv_row(degp_ref[...])
    acc = accpt_ref[0, :, :_N] + accpt_ref[1, :, :_N] + hpt_ref[...]
    x3ot = dinvr * acc + bc_ref[...]
    score = _mm(wpt_ref[...], x3ot)
    xpt = x3ot * (1.0 / (1.0 + jnp.exp(-score)))
    oht, cntr = _onehot_t(batr)
    gapt = _mm_rt(xpt, oht) / jnp.maximum(cntr, 1.0)
    gmpt = _gmp_t(batr, xpt, cntr)
    x3t = jnp.concatenate([gmpt, gapt], axis=0)
    ht = x1t_ref[...] + x2t_ref[...] + x3t
    ht = jnp.maximum(_mm(wl1t_ref[...], ht) + bl1c_ref[...], 0.0)
    ht = jnp.maximum(_mm(wl2t_ref[...], ht) + bl2c_ref[...], 0.0)
    out_ref[...] = _mm(wl3t_ref[...], ht) + bl3c_ref[...]


def _tc_final(accpt, hpt, degp, batr, bc, wpt, x1t, x2t,
              wl1t, bl1c, wl2t, bl2c, wl3t, bl3c):
    return pl.pallas_call(
        _tc_final_body,
        out_shape=jax.ShapeDtypeStruct((1, _G), jnp.float32),
    )(accpt, hpt, degp, batr, bc, wpt, x1t, x2t,
      wl1t, bl1c, wl2t, bl2c, wl3t, bl3c)


def _impl(x, edge_index, batch, W1, b1, W2, b2, W3, b3, wp,
          Wl1, bl1, Wl2, bl2, Wl3, bl3):
    src = edge_index[0]
    dst = edge_index[1]
    npad = _EP - _E
    srcw = jnp.concatenate([src, jnp.zeros((npad,), jnp.int32)]
                           ).reshape(_NW, _NCHUNK, _CH)
    dstw = jnp.concatenate([dst, jnp.full((npad,), _N, jnp.int32)]
                           ).reshape(_NW, _NCHUNK, _CH)
    batr = batch.reshape(1, _N)
    ones128 = jnp.ones((_CH,), jnp.float32)
    zeros1 = jnp.zeros((_NP,), jnp.float32)
    zeros2 = jnp.zeros((_NP, _H), jnp.float32)

    degp = _deg_sc(dstw, ones128, zeros1)

    def pad_rows(ht):
        return jnp.pad(ht.T, ((0, _NP - _N), (0, 0)))

    def tpose(accp):
        return jnp.transpose(accp, (0, 2, 1))

    h1pt = _tc_pre1(x.T, batr, degp, W1[:_D].T, W1[_D:].T)
    acc1 = _msg_sc(pad_rows(h1pt), srcw, dstw, zeros2)
    x1t, h2pt = _tc_mid(tpose(acc1), h1pt, degp, batr, b1.reshape(_H, 1),
                        W2[:_H].T, W2[_H:].T)
    acc2 = _msg_sc(pad_rows(h2pt), srcw, dstw, zeros2)
    x2t, h3pt = _tc_mid(tpose(acc2), h2pt, degp, batr, b2.reshape(_H, 1),
                        W3[:_H].T, W3[_H:].T)
    acc3 = _msg_sc(pad_rows(h3pt), srcw, dstw, zeros2)
    outt = _tc_final(tpose(acc3), h3pt, degp, batr, b3.reshape(_H, 1),
                     wp.T, x1t, x2t, Wl1.T, bl1.reshape(_H, 1),
                     Wl2.T, bl2.reshape(_H // 2, 1), Wl3.T, bl3.reshape(1, 1))
    return outt.reshape(_G, 1)


kernel = _impl
